# baseline (speedup 1.0000x reference)
import functools

import jax
import jax.numpy as jnp
from jax.experimental import pallas as pl
from jax.experimental.pallas import tpu as pltpu


def _se_ring_kernel(x_hbm, w1_ref, b1_ref, w2_ref, b2_ref, o_ref,
                    ring_ref, g_ref, sem, *, nb, inv_t):
    b = pl.program_id(0)

    @pl.when(b == 0)
    def _first():
        pltpu.make_async_copy(x_hbm.at[0], ring_ref.at[0], sem.at[0]).start()

    @pl.when(b + 1 < nb)
    def _prefetch():
        slot = jax.lax.rem(b + 1, 3)
        pltpu.make_async_copy(x_hbm.at[b + 1], ring_ref.at[slot],
                              sem.at[slot]).start()

    @pl.when(b < nb)
    def _gate_cur():
        slot = jax.lax.rem(b, 3)
        pltpu.make_async_copy(ring_ref.at[slot], ring_ref.at[slot],
                              sem.at[slot]).wait()
        x = ring_ref[slot]
        T = x.shape[-1]
        part = x[:, 0:128]
        for k in range(1, T // 128):
            part = part + x[:, k * 128:(k + 1) * 128]
        m = jnp.sum(part, axis=-1, keepdims=True) * inv_t
        h = jnp.dot(w1_ref[...], m,
                    preferred_element_type=jnp.float32) + b1_ref[...]
        h = jnp.maximum(h, 0.0)
        y = jnp.dot(w2_ref[...], h,
                    preferred_element_type=jnp.float32) + b2_ref[...]
        g_ref[jax.lax.rem(b, 2)] = jax.nn.sigmoid(y)

    @pl.when(b > 0)
    def _rescale():
        o_ref[0] = (ring_ref[jax.lax.rem(b - 1, 3)]
                    * g_ref[jax.lax.rem(b - 1, 2)]).astype(o_ref.dtype)


def kernel(x, w1, b1, w2, b2):
    B, C, T = x.shape
    Cs = w1.shape[0]
    const = lambda b: (0, 0)

    return pl.pallas_call(
        functools.partial(_se_ring_kernel, nb=B, inv_t=1.0 / T),
        out_shape=jax.ShapeDtypeStruct((B, C, T), x.dtype),
        grid=(B + 1,),
        in_specs=[
            pl.BlockSpec(memory_space=pl.ANY),
            pl.BlockSpec((Cs, C), const),
            pl.BlockSpec((Cs, 1), const),
            pl.BlockSpec((C, Cs), const),
            pl.BlockSpec((C, 1), const),
        ],
        out_specs=pl.BlockSpec((1, C, T),
                               lambda b: (jnp.maximum(b - 1, 0), 0, 0)),
        scratch_shapes=[
            pltpu.VMEM((3, C, T), x.dtype),
            pltpu.VMEM((2, C, 1), jnp.float32),
            pltpu.SemaphoreType.DMA((3,)),
        ],
        compiler_params=pltpu.CompilerParams(
            dimension_semantics=("arbitrary",),
            vmem_limit_bytes=58 * 1024 * 1024),
    )(x, w1.astype(jnp.float32), b1.reshape(Cs, 1).astype(jnp.float32),
      w2.astype(jnp.float32), b2.reshape(C, 1).astype(jnp.float32))


# speedup vs baseline: 1.0045x; 1.0045x over previous
import functools

import jax
import jax.numpy as jnp
from jax.experimental import pallas as pl
from jax.experimental.pallas import tpu as pltpu


def _se_pipe_kernel(x_ref, w1_ref, b1_ref, w2_ref, b2_ref, o_ref,
                    ping_ref, g_ref, *, nb, inv_t):
    b = pl.program_id(0)

    @pl.when(b < nb)
    def _stage():
        x = x_ref[0].astype(jnp.float32)
        ping_ref[jax.lax.rem(b, 2)] = x
        T = x.shape[-1]
        part = x[:, 0:128]
        for k in range(1, T // 128):
            part = part + x[:, k * 128:(k + 1) * 128]
        m = jnp.sum(part, axis=-1, keepdims=True) * inv_t
        h = jnp.dot(w1_ref[...], m,
                    preferred_element_type=jnp.float32) + b1_ref[...]
        h = jnp.maximum(h, 0.0)
        y = jnp.dot(w2_ref[...], h,
                    preferred_element_type=jnp.float32) + b2_ref[...]
        g_ref[jax.lax.rem(b, 2)] = jax.nn.sigmoid(y)

    @pl.when(b > 0)
    def _rescale():
        o_ref[0] = (ping_ref[jax.lax.rem(b - 1, 2)]
                    * g_ref[jax.lax.rem(b - 1, 2)]).astype(o_ref.dtype)


def kernel(x, w1, b1, w2, b2):
    B, C, T = x.shape
    Cs = w1.shape[0]
    const = lambda b: (0, 0)

    return pl.pallas_call(
        functools.partial(_se_pipe_kernel, nb=B, inv_t=1.0 / T),
        out_shape=jax.ShapeDtypeStruct((B, C, T), x.dtype),
        grid=(B + 1,),
        in_specs=[
            pl.BlockSpec((1, C, T), lambda b: (jnp.minimum(b, B - 1), 0, 0)),
            pl.BlockSpec((Cs, C), const),
            pl.BlockSpec((Cs, 1), const),
            pl.BlockSpec((C, Cs), const),
            pl.BlockSpec((C, 1), const),
        ],
        out_specs=pl.BlockSpec((1, C, T),
                               lambda b: (jnp.maximum(b - 1, 0), 0, 0)),
        scratch_shapes=[
            pltpu.VMEM((2, C, T), jnp.float32),
            pltpu.VMEM((2, C, 1), jnp.float32),
        ],
        compiler_params=pltpu.CompilerParams(
            dimension_semantics=("arbitrary",),
            vmem_limit_bytes=58 * 1024 * 1024),
    )(x, w1.astype(jnp.float32), b1.reshape(Cs, 1).astype(jnp.float32),
      w2.astype(jnp.float32), b2.reshape(C, 1).astype(jnp.float32))


# speedup vs baseline: 1.0054x; 1.0008x over previous
import functools

import jax
import jax.numpy as jnp
from jax.experimental import pallas as pl
from jax.experimental.pallas import tpu as pltpu


def _se_pipe_kernel(x_ref, w1_ref, b1_ref, w2_ref, b2_ref, o_ref,
                    ping_ref, g_ref, *, nb, inv_t):
    b = pl.program_id(0)

    @pl.when(b < nb)
    def _stage():
        x = x_ref[0].astype(jnp.float32)
        ping_ref[jax.lax.rem(b, 2)] = x
        T = x.shape[-1]
        if T % 128 == 0:
            part = x[:, 0:128]
            for k in range(1, T // 128):
                part = part + x[:, k * 128:(k + 1) * 128]
            m = jnp.sum(part, axis=-1, keepdims=True) * inv_t
        else:
            m = jnp.sum(x, axis=-1, keepdims=True) * inv_t

        h = jnp.dot(w1_ref[...], m,
                    preferred_element_type=jnp.float32) + b1_ref[...]
        h = jnp.maximum(h, 0.0)
        y = jnp.dot(w2_ref[...], h,
                    preferred_element_type=jnp.float32) + b2_ref[...]
        g_ref[jax.lax.rem(b, 2)] = jax.nn.sigmoid(y)

    @pl.when(b > 0)
    def _rescale():
        o_ref[0] = (ping_ref[jax.lax.rem(b - 1, 2)]
                    * g_ref[jax.lax.rem(b - 1, 2)]).astype(o_ref.dtype)


def kernel(x, w1, b1, w2, b2):
    B, C, T = x.shape
    Cs = w1.shape[0]
    const = lambda b: (0, 0)

    return pl.pallas_call(
        functools.partial(_se_pipe_kernel, nb=B, inv_t=1.0 / T),
        out_shape=jax.ShapeDtypeStruct((B, C, T), x.dtype),
        grid=(B + 1,),
        in_specs=[
            pl.BlockSpec((1, C, T), lambda b: (jnp.minimum(b, B - 1), 0, 0)),
            pl.BlockSpec((Cs, C), const),
            pl.BlockSpec((Cs, 1), const),
            pl.BlockSpec((C, Cs), const),
            pl.BlockSpec((C, 1), const),
        ],
        out_specs=pl.BlockSpec((1, C, T),
                               lambda b: (jnp.maximum(b - 1, 0), 0, 0)),
        scratch_shapes=[
            pltpu.VMEM((2, C, T), jnp.float32),
            pltpu.VMEM((2, C, 1), jnp.float32),
        ],
        compiler_params=pltpu.CompilerParams(
            dimension_semantics=("arbitrary",),
            vmem_limit_bytes=58 * 1024 * 1024),
    )(x, w1.astype(jnp.float32), b1.reshape(Cs, 1).astype(jnp.float32),
      w2.astype(jnp.float32), b2.reshape(C, 1).astype(jnp.float32))
